# baseline (speedup 1.0000x reference)
import functools

import jax
import jax.numpy as jnp
from jax import lax
from jax.experimental import pallas as pl
from jax.experimental.pallas import tpu as pltpu
from jax.experimental.pallas import tpu_sc as plsc

_S = 8192
_D = 768
_CHUNK_ROWS = 32
_NBUF = 4


@functools.cache
def _make_copy_kernel():
    info = plsc.get_sparse_core_info()
    num_cores, num_subcores = info.num_cores, info.num_subcores
    num_workers = num_cores * num_subcores
    rows_per_worker = _S // num_workers
    num_chunks = rows_per_worker // _CHUNK_ROWS

    mesh = plsc.VectorSubcoreMesh(core_axis_name="c", subcore_axis_name="s")

    scratch = [pltpu.VMEM((_CHUNK_ROWS, _D), jnp.float32) for _ in range(_NBUF)]
    scratch += [pltpu.SemaphoreType.DMA for _ in range(2 * _NBUF)]

    @functools.partial(
        pl.kernel,
        mesh=mesh,
        out_type=jax.ShapeDtypeStruct((_S, _D), jnp.float32),
        scratch_types=scratch,
    )
    def copy_k(table_hbm, out_hbm, *scratch_refs):
        bufs = scratch_refs[:_NBUF]
        in_sems = scratch_refs[_NBUF : 2 * _NBUF]
        out_sems = scratch_refs[2 * _NBUF :]

        wid = lax.axis_index("s") * num_cores + lax.axis_index("c")
        base = wid * rows_per_worker

        def start_in(i):
            return pltpu.async_copy(
                table_hbm.at[pl.ds(base + i * _CHUNK_ROWS, _CHUNK_ROWS)],
                bufs[i % _NBUF],
                in_sems[i % _NBUF],
            )

        def start_out(i):
            return pltpu.async_copy(
                bufs[i % _NBUF],
                out_hbm.at[pl.ds(base + i * _CHUNK_ROWS, _CHUNK_ROWS)],
                out_sems[i % _NBUF],
            )

        in_cp = [None] * num_chunks
        out_cp = [None] * num_chunks
        in_cp[0] = start_in(0)
        for i in range(num_chunks):
            in_cp[i].wait()
            if i + 1 < num_chunks:
                if i + 1 >= _NBUF:
                    out_cp[i + 1 - _NBUF].wait()
                in_cp[i + 1] = start_in(i + 1)
            out_cp[i] = start_out(i)
        for i in range(max(0, num_chunks - _NBUF), num_chunks):
            out_cp[i].wait()

    return copy_k


def kernel(x, pos_table):
    del x
    return _make_copy_kernel()(pos_table)[None]


# speedup vs baseline: 1.0180x; 1.0180x over previous
import functools

import jax
import jax.numpy as jnp
from jax import lax
from jax.experimental import pallas as pl
from jax.experimental.pallas import tpu as pltpu
from jax.experimental.pallas import tpu_sc as plsc

_S = 8192
_D = 768
_CHUNK_ROWS = 64
_NBUF = 2
_CHUNK_SIZES = (16, 48, 64, 64, 64)


@functools.cache
def _make_copy_kernel():
    info = plsc.get_sparse_core_info()
    num_cores, num_subcores = info.num_cores, info.num_subcores
    num_workers = num_cores * num_subcores
    rows_per_worker = _S // num_workers
    assert sum(_CHUNK_SIZES) == rows_per_worker
    num_chunks = len(_CHUNK_SIZES)
    offsets = [sum(_CHUNK_SIZES[:i]) for i in range(num_chunks)]

    mesh = plsc.VectorSubcoreMesh(core_axis_name="c", subcore_axis_name="s")

    scratch = [pltpu.VMEM((_CHUNK_ROWS, _D), jnp.float32) for _ in range(_NBUF)]
    scratch += [pltpu.SemaphoreType.DMA for _ in range(2 * _NBUF)]

    @functools.partial(
        pl.kernel,
        mesh=mesh,
        out_type=jax.ShapeDtypeStruct((_S, _D), jnp.float32),
        scratch_types=scratch,
    )
    def copy_k(table_hbm, out_hbm, *scratch_refs):
        bufs = scratch_refs[:_NBUF]
        in_sems = scratch_refs[_NBUF : 2 * _NBUF]
        out_sems = scratch_refs[2 * _NBUF :]

        wid = lax.axis_index("s") * num_cores + lax.axis_index("c")
        base = wid * rows_per_worker

        def start_in(i):
            n = _CHUNK_SIZES[i]
            return pltpu.async_copy(
                table_hbm.at[pl.ds(base + offsets[i], n)],
                bufs[i % _NBUF].at[pl.ds(0, n)],
                in_sems[i % _NBUF],
            )

        def start_out(i):
            n = _CHUNK_SIZES[i]
            return pltpu.async_copy(
                bufs[i % _NBUF].at[pl.ds(0, n)],
                out_hbm.at[pl.ds(base + offsets[i], n)],
                out_sems[i % _NBUF],
            )

        in_cp = [None] * num_chunks
        out_cp = [None] * num_chunks
        in_cp[0] = start_in(0)
        for i in range(num_chunks):
            in_cp[i].wait()
            if i + 1 < num_chunks:
                if i + 1 >= _NBUF:
                    out_cp[i + 1 - _NBUF].wait()
                in_cp[i + 1] = start_in(i + 1)
            out_cp[i] = start_out(i)
        for i in range(max(0, num_chunks - _NBUF), num_chunks):
            out_cp[i].wait()

    return copy_k


def kernel(x, pos_table):
    del x
    return _make_copy_kernel()(pos_table)[None]


# speedup vs baseline: 1.0403x; 1.0218x over previous
import functools

import jax
import jax.numpy as jnp
from jax import lax
from jax.experimental import pallas as pl
from jax.experimental.pallas import tpu as pltpu
from jax.experimental.pallas import tpu_sc as plsc

_S = 8192
_D = 768
_NBUF = 2
_CHUNK_SIZES = (64, 64, 64, 64)


@functools.cache
def _make_copy_kernel():
    info = plsc.get_sparse_core_info()
    num_cores, num_subcores = info.num_cores, info.num_subcores
    num_workers = num_cores * num_subcores
    rows_per_worker = _S // num_workers
    assert sum(_CHUNK_SIZES) == rows_per_worker
    num_chunks = len(_CHUNK_SIZES)
    offsets = [sum(_CHUNK_SIZES[:i]) for i in range(num_chunks)]

    mesh = plsc.VectorSubcoreMesh(core_axis_name="c", subcore_axis_name="s")

    scratch = [pltpu.VMEM((max(_CHUNK_SIZES), _D), jnp.float32) for _ in range(_NBUF)]
    scratch += [pltpu.SemaphoreType.DMA for _ in range(2 * _NBUF)]

    @functools.partial(
        pl.kernel,
        mesh=mesh,
        out_type=jax.ShapeDtypeStruct((_S, _D), jnp.float32),
        scratch_types=scratch,
    )
    def copy_k(table_hbm, out_hbm, *scratch_refs):
        bufs = scratch_refs[:_NBUF]
        in_sems = scratch_refs[_NBUF : 2 * _NBUF]
        out_sems = scratch_refs[2 * _NBUF :]

        wid = lax.axis_index("s") * num_cores + lax.axis_index("c")
        base = wid * rows_per_worker

        def start_in(i):
            n = _CHUNK_SIZES[i]
            return pltpu.async_copy(
                table_hbm.at[pl.ds(base + offsets[i], n)],
                bufs[i % _NBUF].at[pl.ds(0, n)],
                in_sems[i % _NBUF],
            )

        def start_out(i):
            n = _CHUNK_SIZES[i]
            return pltpu.async_copy(
                bufs[i % _NBUF].at[pl.ds(0, n)],
                out_hbm.at[pl.ds(base + offsets[i], n)],
                out_sems[i % _NBUF],
            )

        in_cp = [None] * num_chunks
        out_cp = [None] * num_chunks
        in_cp[0] = start_in(0)
        for i in range(num_chunks):
            in_cp[i].wait()
            if i + 1 < num_chunks:
                if i + 1 >= _NBUF:
                    out_cp[i + 1 - _NBUF].wait()
                in_cp[i + 1] = start_in(i + 1)
            out_cp[i] = start_out(i)
        for i in range(max(0, num_chunks - _NBUF), num_chunks):
            out_cp[i].wait()

    return copy_k


def kernel(x, pos_table):
    del x
    return _make_copy_kernel()(pos_table)[None]
